# baseline (speedup 1.0000x reference)
import functools

import jax
import jax.numpy as jnp
from jax import lax
from jax.experimental import pallas as pl
from jax.experimental.pallas import tpu as pltpu
from jax.experimental.pallas import tpu_sc as plsc

NC = 2
NS = 16
NW = NC * NS


def _sc_gather_scatter_add(table, idx4, zrows):
    N, D = table.shape
    _, NCH, _, C = idx4.shape
    NB = 4
    assert NCH % NB == 0 and NCH >= 3 * NB
    NACC = N + 8
    RA = ((N // NS) + 7) // 8 * 8
    RL = N - RA * (NS - 1)
    assert RL > 0 and RA % 8 == 0 and RL % 8 == 0
    mesh = plsc.VectorSubcoreMesh(core_axis_name="c", subcore_axis_name="s")

    @functools.partial(
        pl.kernel,
        out_type=jax.ShapeDtypeStruct((NC, N, D), jnp.float32),
        mesh=mesh,
        scratch_types=[
            [pltpu.VMEM((2, C), jnp.int32) for _ in range(NB)],
            [pltpu.VMEM((C, D), jnp.float32) for _ in range(NB)],
            pltpu.VMEM_SHARED((NACC, D), jnp.float32),
            [pltpu.SemaphoreType.DMA for _ in range(NB)],
            [pltpu.SemaphoreType.DMA for _ in range(NB)],
            [pltpu.SemaphoreType.DMA for _ in range(NB)],
        ],
    )
    def k(table_hbm, idx_hbm, z_hbm, out_hbm,
          ibuf, rows, acc, isem, gsem, ssem):
        c = lax.axis_index("c")
        s = lax.axis_index("s")
        wid = c * NS + s

        @pl.when(s < NS - 1)
        def _():
            pltpu.sync_copy(z_hbm, acc.at[pl.ds(s * RA, RA)])

        @pl.when(s == NS - 1)
        def _():
            pltpu.sync_copy(z_hbm.at[pl.ds(0, RL)],
                            acc.at[pl.ds((NS - 1) * RA, RL)])

        plsc.subcore_barrier()

        def issue_idx(j, b):
            pltpu.async_copy(idx_hbm.at[wid, j], ibuf[b], isem[b])

        def issue_gather(j_unused, b, rb):
            pltpu.async_copy(table_hbm.at[ibuf[b].at[0]], rows[rb], gsem[rb])

        def issue_scatter(j_unused, b):
            pltpu.async_copy(rows[b], acc.at[ibuf[b].at[1]], ssem[b],
                             add=True)

        def wait_idx(b):
            pltpu.make_async_copy(idx_hbm.at[0, 0], ibuf[b], isem[b]).wait()

        def wait_gather(b):
            pltpu.make_async_copy(table_hbm.at[pl.ds(0, C)], rows[b],
                                  gsem[b]).wait()

        def wait_scatter(b):
            pltpu.make_async_copy(rows[b], acc.at[pl.ds(0, C)],
                                  ssem[b]).wait()

        def steady(j, bj, *, first=False, idx_ahead=True, gather_ahead=True):
            if not first:
                wait_scatter((bj - 1) % NB)
            if idx_ahead:
                issue_idx(j + (NB - 1), (bj - 1) % NB)
            if gather_ahead:
                wait_idx((bj + 1) % NB)
                issue_gather(j + 1, (bj + 1) % NB, (bj + 1) % NB)
            wait_gather(bj)
            issue_scatter(j, bj)

        issue_idx(0, 0)
        issue_idx(1, 1)
        issue_idx(2, 2)
        wait_idx(0)
        issue_gather(0, 0, 0)
        steady(0, 0, first=True)

        @pl.loop(0, (NCH - NB) // NB)
        def _(p):
            for b in range(NB):
                steady(NB * p + 1 + b, (1 + b) % NB)

        steady(NCH - 3, (NCH - 3) % NB, idx_ahead=False)
        steady(NCH - 2, (NCH - 2) % NB, idx_ahead=False)
        steady(NCH - 1, (NCH - 1) % NB, idx_ahead=False, gather_ahead=False)
        wait_scatter((NCH - 1) % NB)

        plsc.subcore_barrier()

        @pl.when(s < NS - 1)
        def _():
            pltpu.sync_copy(acc.at[pl.ds(s * RA, RA)],
                            out_hbm.at[c, pl.ds(s * RA, RA)])

        @pl.when(s == NS - 1)
        def _():
            pltpu.sync_copy(acc.at[pl.ds((NS - 1) * RA, RL)],
                            out_hbm.at[c, pl.ds((NS - 1) * RA, RL)])

    return k(table, idx4, zrows)


def _tc_layer(a0, a1, feats, W, b, lam):
    N, D = feats.shape
    R = 1000
    G = N // R

    def body(a0_ref, a1_ref, f_ref, w_ref, b_ref, lam_ref, o_ref):
        z = a0_ref[...] + a1_ref[...] + lam_ref[0, 0] * f_ref[...]
        y = jnp.dot(z, w_ref[...], preferred_element_type=jnp.float32)
        o_ref[...] = jax.nn.sigmoid(y + b_ref[...])

    return pl.pallas_call(
        body,
        grid=(G,),
        in_specs=[
            pl.BlockSpec((R, D), lambda i: (i, 0)),
            pl.BlockSpec((R, D), lambda i: (i, 0)),
            pl.BlockSpec((R, D), lambda i: (i, 0)),
            pl.BlockSpec((D, D), lambda i: (0, 0)),
            pl.BlockSpec((1, D), lambda i: (0, 0)),
            pl.BlockSpec((1, 1), lambda i: (0, 0)),
        ],
        out_specs=pl.BlockSpec((R, D), lambda i: (i, 0)),
        out_shape=jax.ShapeDtypeStruct((N, D), jnp.float32),
    )(a0, a1, feats, W, b, lam)


def _tc_layer_final(a0, a1, feats, W, b, lam, fcw_row, fcb):
    N, D = feats.shape
    R = 1000
    G = N // R

    def body(a0_ref, a1_ref, f_ref, w_ref, b_ref, lam_ref, fcw_ref, fcb_ref,
             out_ref, gr_ref):
        i = pl.program_id(0)
        z = a0_ref[...] + a1_ref[...] + lam_ref[0, 0] * f_ref[...]
        y = jax.nn.sigmoid(
            jnp.dot(z, w_ref[...], preferred_element_type=jnp.float32)
            + b_ref[...])

        @pl.when(i == 0)
        def _():
            gr_ref[...] = jnp.zeros_like(gr_ref)

        gr_ref[...] += jnp.sum(y, axis=0, keepdims=True)

        @pl.when(i == G - 1)
        def _():
            out_ref[...] = (jnp.sum(gr_ref[...] * fcw_ref[...], axis=1,
                                    keepdims=True) + fcb_ref[...])

    blk = lambda r, c: pl.BlockSpec((r, c), lambda i: (i, 0))
    const = lambda r, c: pl.BlockSpec((r, c), lambda i: (0, 0))
    out, gr = pl.pallas_call(
        body,
        grid=(G,),
        in_specs=[
            blk(R, D), blk(R, D), blk(R, D),
            const(D, D), const(1, D), const(1, 1),
            const(1, D), const(1, 1),
        ],
        out_specs=[const(1, 1), const(1, D)],
        out_shape=[
            jax.ShapeDtypeStruct((1, 1), jnp.float32),
            jax.ShapeDtypeStruct((1, D), jnp.float32),
        ],
    )(a0, a1, feats, W, b, lam, fcw_row, fcb)
    return out, gr


def kernel(x, edge_index, W1, b1, adj1, W2, b2, adj2, fc_W, fc_b):
    N, D = x.shape
    src = edge_index[0].astype(jnp.int32)
    dst = edge_index[1].astype(jnp.int32)
    zrows = jnp.zeros((((N // NS) + 7) // 8 * 8, D), jnp.float32)
    b1r = b1.reshape(1, D)
    b2r = b2.reshape(1, D)
    lam1 = adj1.reshape(1, 1).astype(jnp.float32)
    lam2 = adj2.reshape(1, 1).astype(jnp.float32)
    fcw_row = fc_W.reshape(1, D)
    fcb = fc_b.reshape(1, 1)

    C = 80
    EPW = src.shape[0] // NW
    NCH = (EPW + C - 1) // C
    NCH = (NCH + 3) // 4 * 4
    CAP = NCH * C
    srcw = jnp.pad(src.reshape(NW, EPW), ((0, 0), (0, CAP - EPW)),
                   constant_values=0)
    dstw = jnp.pad(dst.reshape(NW, EPW), ((0, 0), (0, CAP - EPW)),
                   constant_values=N)
    idx4 = jnp.concatenate(
        [srcw.reshape(NW, NCH, 1, C), dstw.reshape(NW, NCH, 1, C)], axis=2)

    p1 = _sc_gather_scatter_add(x, idx4, zrows)
    f1 = _tc_layer(p1[0], p1[1], x, W1, b1r, lam1)
    p2 = _sc_gather_scatter_add(f1, idx4, zrows)
    out, gr = _tc_layer_final(p2[0], p2[1], f1, W2, b2r, lam2, fcw_row, fcb)
    return (out, gr)
